# baseline (speedup 1.0000x reference)
import functools

import jax
import jax.numpy as jnp
from jax import lax
from jax.experimental import pallas as pl
from jax.experimental.pallas import tpu as pltpu
from jax.experimental.pallas import tpu_sc as plsc

_NC = 2
_NS = 16
_NW = _NC * _NS
_L = 16
_CH = 128
_D = 128


def _mesh():
    return plsc.VectorSubcoreMesh(
        core_axis_name="c", subcore_axis_name="s", num_cores=_NC, num_subcores=_NS
    )


def _dyn_splat(vec16, lane):
    idx = jnp.full((_L, 1), lane, jnp.int32)
    dn = lax.GatherDimensionNumbers(
        offset_dims=(), collapsed_slice_dims=(0,), start_index_map=(0,)
    )
    return lax.gather(
        vec16, idx, dn, (1,), mode=lax.GatherScatterMode.PROMISE_IN_BOUNDS
    )


def _make_deg_kernel(tch, npad):
    strip = npad // _NS

    @functools.partial(
        pl.kernel,
        out_type=jax.ShapeDtypeStruct((_NC, npad), jnp.float32),
        mesh=_mesh(),
        scratch_types=[
            pltpu.VMEM((tch, _CH), jnp.int32),
            pltpu.VMEM((tch, _CH), jnp.float32),
            pltpu.VMEM((strip,), jnp.float32),
            pltpu.VMEM_SHARED((npad,), jnp.float32),
        ],
    )
    def deg_kernel(dst_hbm, w_hbm, out_hbm, dstv, wv, buf, deg_sh):
        cid = lax.axis_index("c")
        sid = lax.axis_index("s")
        wid = sid * _NC + cid
        z = jnp.zeros((_L,), jnp.float32)

        def zv(r, _):
            buf[pl.ds(r * _L, _L)] = z
            return 0

        lax.fori_loop(0, strip // _L, zv, 0)
        pltpu.sync_copy(buf, deg_sh.at[pl.ds(sid * strip, strip)])
        pltpu.sync_copy(dst_hbm.at[wid], dstv)
        pltpu.sync_copy(w_hbm.at[wid], wv)
        plsc.subcore_barrier()

        def chunk(j, _):
            pltpu.sync_copy(wv.at[j], deg_sh.at[dstv.at[j]], add=True)
            return 0

        lax.fori_loop(0, tch, chunk, 0)
        plsc.subcore_barrier()
        pltpu.sync_copy(deg_sh.at[pl.ds(sid * strip, strip)], buf)
        pltpu.sync_copy(buf, out_hbm.at[cid, pl.ds(sid * strip, strip)])

    return deg_kernel


def _make_acc_kernel(tch, npad, nrows):
    strip = npad // _NS

    @functools.partial(
        pl.kernel,
        out_type=jax.ShapeDtypeStruct((_NC, npad, _D), jnp.float32),
        mesh=_mesh(),
        scratch_types=[
            pltpu.VMEM((tch, _CH), jnp.int32),
            pltpu.VMEM((tch, _CH), jnp.int32),
            pltpu.VMEM((tch, _CH), jnp.float32),
            pltpu.VMEM((_CH, _D), jnp.float32),
            pltpu.VMEM_SHARED((npad, _D), jnp.float32),
            pltpu.SemaphoreType.DMA,
        ],
    )
    def acc_kernel(y_hbm, src_hbm, dst_hbm, w_hbm, out_hbm,
                   srcv, dstv, wv, rows, acc_sh, sem):
        cid = lax.axis_index("c")
        sid = lax.axis_index("s")
        wid = sid * _NC + cid
        z = jnp.zeros((_L,), jnp.float32)

        def zrow(r, _):
            for q in range(_D // _L):
                rows[r, pl.ds(q * _L, _L)] = z
            return 0

        lax.fori_loop(0, _CH, zrow, 0)
        for b in range(strip // _CH):
            pltpu.sync_copy(rows, acc_sh.at[pl.ds(sid * strip + b * _CH, _CH)])

        pltpu.sync_copy(src_hbm.at[wid], srcv)
        pltpu.sync_copy(dst_hbm.at[wid], dstv)
        pltpu.sync_copy(w_hbm.at[wid], wv)
        plsc.subcore_barrier()

        def chunk(j, _):
            pltpu.async_copy(y_hbm.at[srcv.at[j]], rows, sem).wait()

            def erow(e, _):
                g = lax.shift_right_logical(e, 4)
                w16 = wv[j, pl.ds(g * _L, _L)]
                sp = _dyn_splat(w16, e & (_L - 1))
                for q in range(_D // _L):
                    rows[e, pl.ds(q * _L, _L)] = rows[e, pl.ds(q * _L, _L)] * sp
                return 0

            lax.fori_loop(0, _CH, erow, 0)
            pltpu.sync_copy(rows, acc_sh.at[dstv.at[j]], add=True)
            return 0

        lax.fori_loop(0, tch, chunk, 0)
        plsc.subcore_barrier()
        pltpu.sync_copy(acc_sh.at[pl.ds(sid * strip, strip)],
                        out_hbm.at[cid, pl.ds(sid * strip, strip)])

    return acc_kernel


_BR = 1000


def _tc_stage1(x, W1, degp):
    n = x.shape[0]

    def body(xr, wr, dr, yr, disr):
        deg = dr[0] + dr[1] + 1.0
        dis = lax.rsqrt(deg)
        xw = jnp.dot(xr[...], wr[...], preferred_element_type=jnp.float32)
        yr[...] = xw * dis
        disr[...] = dis

    return pl.pallas_call(
        body,
        grid=(n // _BR,),
        in_specs=[
            pl.BlockSpec((_BR, _D), lambda i: (i, 0)),
            pl.BlockSpec((_D, _D), lambda i: (0, 0)),
            pl.BlockSpec((_NC, _BR, 1), lambda i: (0, i, 0)),
        ],
        out_specs=[
            pl.BlockSpec((_BR, _D), lambda i: (i, 0)),
            pl.BlockSpec((_BR, 1), lambda i: (i, 0)),
        ],
        out_shape=[
            jax.ShapeDtypeStruct((n, _D), jnp.float32),
            jax.ShapeDtypeStruct((n, 1), jnp.float32),
        ],
    )(x, W1, degp)


def _tc_stage2(acc, y1, dis, b1, W2):
    n = y1.shape[0]

    def body(ar, yr, dr, br, wr, outr):
        h = jax.nn.relu(dr[...] * (ar[0] + ar[1] + yr[...]) + br[...])
        hw = jnp.dot(h, wr[...], preferred_element_type=jnp.float32)
        outr[...] = hw * dr[...]

    return pl.pallas_call(
        body,
        grid=(n // _BR,),
        in_specs=[
            pl.BlockSpec((_NC, _BR, _D), lambda i: (0, i, 0)),
            pl.BlockSpec((_BR, _D), lambda i: (i, 0)),
            pl.BlockSpec((_BR, 1), lambda i: (i, 0)),
            pl.BlockSpec((1, _D), lambda i: (0, 0)),
            pl.BlockSpec((_D, _D), lambda i: (0, 0)),
        ],
        out_specs=pl.BlockSpec((_BR, _D), lambda i: (i, 0)),
        out_shape=jax.ShapeDtypeStruct((n, _D), jnp.float32),
    )(acc, y1, dis, b1.reshape(1, _D), W2)


def _tc_stage3(acc, y2, dis, b2, Wmu, bmu, Wlv, blv):
    n = y2.shape[0]
    lout = Wmu.shape[1]
    nblk = n // _BR

    def body(ar, yr, dr, br, wmr, bmr, wlr, blr, mur, lvr, scr):
        i = pl.program_id(0)

        @pl.when(i == 0)
        def _():
            scr[...] = jnp.zeros_like(scr)

        out2 = dr[...] * (ar[0] + ar[1] + yr[...])
        scr[...] += jnp.sum(out2, axis=0, keepdims=True)

        @pl.when(i == nblk - 1)
        def _():
            g = scr[...] * (1.0 / n) + br[...]
            mur[...] = jnp.dot(g, wmr[...], preferred_element_type=jnp.float32) + bmr[...]
            lvr[...] = jnp.dot(g, wlr[...], preferred_element_type=jnp.float32) + blr[...]

    return pl.pallas_call(
        body,
        grid=(nblk,),
        in_specs=[
            pl.BlockSpec((_NC, _BR, _D), lambda i: (0, i, 0)),
            pl.BlockSpec((_BR, _D), lambda i: (i, 0)),
            pl.BlockSpec((_BR, 1), lambda i: (i, 0)),
            pl.BlockSpec((1, _D), lambda i: (0, 0)),
            pl.BlockSpec((_D, lout), lambda i: (0, 0)),
            pl.BlockSpec((1, lout), lambda i: (0, 0)),
            pl.BlockSpec((_D, lout), lambda i: (0, 0)),
            pl.BlockSpec((1, lout), lambda i: (0, 0)),
        ],
        out_specs=[
            pl.BlockSpec((1, lout), lambda i: (0, 0)),
            pl.BlockSpec((1, lout), lambda i: (0, 0)),
        ],
        out_shape=[
            jax.ShapeDtypeStruct((1, lout), jnp.float32),
            jax.ShapeDtypeStruct((1, lout), jnp.float32),
        ],
        scratch_shapes=[pltpu.VMEM((1, _D), jnp.float32)],
    )(acc, y2, dis, b2.reshape(1, _D), Wmu, bmu.reshape(1, lout), Wlv,
      blv.reshape(1, lout))


def kernel(x, edge_index, weights, W1, b1, W2, b2, Wmu, bmu, Wlv, blv):
    n = x.shape[0]
    e = weights.shape[0]

    npad = ((n + _NS * _CH - 1) // (_NS * _CH)) * (_NS * _CH)
    tch = (e + _NW * _CH - 1) // (_NW * _CH)
    epad = _NW * tch * _CH - e

    zpad_i = jnp.zeros((epad,), jnp.int32)
    src3 = jnp.concatenate([edge_index[0], zpad_i]).reshape(_NW, tch, _CH)
    dst3 = jnp.concatenate([edge_index[1], zpad_i]).reshape(_NW, tch, _CH)
    w3 = jnp.concatenate(
        [weights, jnp.zeros((epad,), jnp.float32)]
    ).reshape(_NW, tch, _CH)

    degp = _make_deg_kernel(tch, npad)(dst3, w3)
    degp3 = degp.reshape(_NC, npad, 1)

    acc_fn = _make_acc_kernel(tch, npad, n)
    y1, dis = _tc_stage1(x, W1, degp3)
    acc1 = acc_fn(y1, src3, dst3, w3)
    y2 = _tc_stage2(acc1, y1, dis, b1, W2)
    acc2 = acc_fn(y2, src3, dst3, w3)
    mu, logvar = _tc_stage3(acc2, y2, dis, b2, Wmu, bmu, Wlv, blv)
    return (mu, logvar)
